# baseline (speedup 1.0000x reference)
import functools

import jax
import jax.numpy as jnp
from jax import lax
from jax.experimental import pallas as pl
from jax.experimental.pallas import tpu as pltpu
from jax.experimental.pallas import tpu_sc as plsc

_LANES = 16
_CHUNK = 16
_NBUF = 4


@functools.partial(jax.jit, static_argnums=(3, 4, 5))
def _sc_expand(input_len, table2, pos_table, B, T, D):
    NC = 2
    NS = 16
    NW = NC * NS
    C = _CHUNK
    G = (B * T) // C
    gpb = T // C
    my_chunks = G // NW

    mesh = plsc.VectorSubcoreMesh(core_axis_name="c", subcore_axis_name="s")

    @functools.partial(
        pl.kernel,
        mesh=mesh,
        out_type=jax.ShapeDtypeStruct((B * T, D), jnp.float32),
        scratch_types=[
            pltpu.VMEM((_LANES,), jnp.int32),
            pltpu.VMEM((C, D), jnp.float32),
            [pltpu.VMEM((C, D), jnp.float32)] * _NBUF,
            pltpu.SemaphoreType.DMA,
            pltpu.SemaphoreType.DMA,
            [pltpu.SemaphoreType.DMA] * _NBUF,
            [pltpu.SemaphoreType.DMA] * _NBUF,
        ],
    )
    def _k(len_hbm, tab_hbm, pos_hbm, out_hbm, lens_v, zbuf, bufs,
           semZ, semB, semG, semS):
        c = lax.axis_index("c")
        s = lax.axis_index("s")
        wid = s * NC + c

        pltpu.sync_copy(len_hbm, lens_v.at[pl.ds(0, B)])
        lens16 = lens_v[...]

        def chunk_m(j):
            g = wid + NW * j
            t0 = (g % gpb) * C
            b = g // gpb
            len_b = lens16[0]
            for bb in range(1, B):
                len_b = jnp.where(b == bb, lens16[bb], len_b)
            return t0, jnp.clip(len_b - t0, 0, C), g * C

        def cnt(j, acc):
            _, m, _ = chunk_m(j)
            return acc + jnp.where(m == 0, 1, 0)

        npad = lax.fori_loop(0, my_chunks, cnt, 0)

        @pl.when(npad > 0)
        def _pads():
            zidx = jnp.zeros((_LANES,), jnp.int32)
            pltpu.async_copy(pos_hbm.at[zidx], zbuf, semB).wait()

            def fire(j, carry):
                _, m, o0 = chunk_m(j)

                @pl.when(m == 0)
                def _():
                    pltpu.make_async_copy(
                        zbuf, out_hbm.at[pl.ds(o0, C)], semZ).start()
                return carry

            lax.fori_loop(0, my_chunks, fire, 0)

        def ring(i, inflight):
            infos = [chunk_m(_NBUF * i + k) for k in range(_NBUF)]
            for k in range(_NBUF):
                t0, m, o0 = infos[k]

                def _fire(k=k, t0=t0, o0=o0, fl=inflight[k]):
                    @pl.when(fl == 1)
                    def _():
                        pltpu.make_async_copy(
                            bufs[k], out_hbm.at[pl.ds(o0, C)], semS[k]).wait()
                    pltpu.make_async_copy(
                        tab_hbm.at[pl.ds(t0, C)], bufs[k], semG[k]).start()

                pl.when(m == C)(_fire)

            for k in range(_NBUF):
                t0, m, o0 = infos[k]

                def _store(k=k, t0=t0, o0=o0):
                    pltpu.make_async_copy(
                        tab_hbm.at[pl.ds(t0, C)], bufs[k], semG[k]).wait()
                    pltpu.make_async_copy(
                        bufs[k], out_hbm.at[pl.ds(o0, C)], semS[k]).start()

                pl.when(m == C)(_store)

            return tuple(
                jnp.where(infos[k][1] == C, jnp.int32(1), inflight[k])
                for k in range(_NBUF))

        inflight = lax.fori_loop(
            0, my_chunks // _NBUF, ring, (jnp.int32(0),) * _NBUF)

        for k in range(_NBUF):
            def _drain(k=k):
                pltpu.make_async_copy(
                    bufs[k], out_hbm.at[pl.ds(wid * C, C)], semS[k]).wait()

            pl.when(inflight[k] == 1)(_drain)

        def boundary(j, carry):
            t0, m, o0 = chunk_m(j)

            @pl.when((m > 0) & (m < C))
            def _():
                r_vec = lax.iota(jnp.int32, _LANES)
                idx = jnp.where(r_vec < m, t0 + 1 + r_vec, 0)
                pltpu.async_copy(pos_hbm.at[idx], bufs[0], semB).wait()
                pltpu.sync_copy(bufs[0], out_hbm.at[pl.ds(o0, C)])
            return carry

        lax.fori_loop(0, my_chunks, boundary, 0)

        def drain(j, carry):
            pltpu.make_async_copy(
                zbuf, out_hbm.at[pl.ds(wid * C, C)], semZ).wait()
            return carry

        lax.fori_loop(0, npad, drain, 0)

    return _k(input_len, table2, pos_table)


def _shift_body(a_ref, b_ref, o_ref):
    o_ref[...] = jnp.concatenate([a_ref[1:], b_ref[:1]], axis=0)


@jax.jit
def _shift_table(pos_table):
    V, D = pos_table.shape
    T = V - 1
    CB = 2048
    return pl.pallas_call(
        _shift_body,
        grid=(T // CB,),
        in_specs=[
            pl.BlockSpec((CB, D), lambda r: (r, 0)),
            pl.BlockSpec((8, D), lambda r: ((r + 1) * (CB // 8), 0)),
        ],
        out_specs=pl.BlockSpec((CB, D), lambda r: (r, 0)),
        out_shape=jax.ShapeDtypeStruct((T, D), jnp.float32),
    )(pos_table, pos_table)


def kernel(input_len, max_len, pos_table):
    del max_len
    V, D = pos_table.shape
    T = V - 1
    B = input_len.shape[0]
    table2 = _shift_table(pos_table)
    out = _sc_expand(input_len, table2, pos_table, B, T, D)
    return out.reshape(B, T, D)


# speedup vs baseline: 1.3564x; 1.3564x over previous
import functools

import jax
import jax.numpy as jnp
from jax import lax
from jax.experimental import pallas as pl
from jax.experimental.pallas import tpu as pltpu
from jax.experimental.pallas import tpu_sc as plsc

_LANES = 16
_CHUNK = 16
_NBUF = 4


@functools.partial(jax.jit, static_argnums=(2, 3, 4))
def _sc_expand(input_len, table2z, B, T, D):
    NC = 2
    NS = 16
    NW = NC * NS
    C = _CHUNK
    G = (B * T) // C
    gpb = T // C
    my_chunks = G // NW

    mesh = plsc.VectorSubcoreMesh(core_axis_name="c", subcore_axis_name="s")

    @functools.partial(
        pl.kernel,
        mesh=mesh,
        out_type=jax.ShapeDtypeStruct((B * T, D), jnp.float32),
        scratch_types=[
            pltpu.VMEM((_LANES,), jnp.int32),
            pltpu.VMEM((C, D), jnp.float32),
            [pltpu.VMEM((C, D), jnp.float32)] * _NBUF,
            pltpu.SemaphoreType.DMA,
            pltpu.SemaphoreType.DMA,
            [pltpu.SemaphoreType.DMA] * _NBUF,
            [pltpu.SemaphoreType.DMA] * _NBUF,
        ],
    )
    def _k(len_hbm, tab_hbm, out_hbm, lens_v, zbuf, bufs,
           semZ, semB, semG, semS):
        c = lax.axis_index("c")
        s = lax.axis_index("s")
        wid = s * NC + c

        pltpu.sync_copy(len_hbm, lens_v.at[pl.ds(0, B)])
        lens16 = lens_v[...]

        def chunk_m(j):
            g = wid + NW * j
            t0 = (g % gpb) * C
            b = g // gpb
            len_b = lens16[0]
            for bb in range(1, B):
                len_b = jnp.where(b == bb, lens16[bb], len_b)
            return t0, jnp.clip(len_b - t0, 0, C), g * C

        def cnt(j, acc):
            _, m, _ = chunk_m(j)
            return acc + jnp.where(m == 0, 1, 0)

        npad = lax.fori_loop(0, my_chunks, cnt, 0)

        @pl.when(npad > 0)
        def _pads():
            pltpu.sync_copy(tab_hbm.at[pl.ds(T, C)], zbuf)

            def fire(j, carry):
                _, m, o0 = chunk_m(j)

                @pl.when(m == 0)
                def _():
                    pltpu.make_async_copy(
                        zbuf, out_hbm.at[pl.ds(o0, C)], semZ).start()
                return carry

            lax.fori_loop(0, my_chunks, fire, 0)

        def ring(i, inflight):
            infos = [chunk_m(_NBUF * i + k) for k in range(_NBUF)]
            for k in range(_NBUF):
                t0, m, o0 = infos[k]

                def _fire(k=k, t0=t0, o0=o0, fl=inflight[k]):
                    @pl.when(fl == 1)
                    def _():
                        pltpu.make_async_copy(
                            bufs[k], out_hbm.at[pl.ds(o0, C)], semS[k]).wait()
                    pltpu.make_async_copy(
                        tab_hbm.at[pl.ds(t0, C)], bufs[k], semG[k]).start()

                pl.when(m == C)(_fire)

            for k in range(_NBUF):
                t0, m, o0 = infos[k]

                def _store(k=k, t0=t0, o0=o0):
                    pltpu.make_async_copy(
                        tab_hbm.at[pl.ds(t0, C)], bufs[k], semG[k]).wait()
                    pltpu.make_async_copy(
                        bufs[k], out_hbm.at[pl.ds(o0, C)], semS[k]).start()

                pl.when(m == C)(_store)

            return tuple(
                jnp.where(infos[k][1] == C, jnp.int32(1), inflight[k])
                for k in range(_NBUF))

        inflight = lax.fori_loop(
            0, my_chunks // _NBUF, ring, (jnp.int32(0),) * _NBUF)

        for k in range(_NBUF):
            def _drain(k=k):
                pltpu.make_async_copy(
                    bufs[k], out_hbm.at[pl.ds(wid * C, C)], semS[k]).wait()

            pl.when(inflight[k] == 1)(_drain)

        def boundary(j, carry):
            t0, m, o0 = chunk_m(j)

            @pl.when((m > 0) & (m < C))
            def _():
                r_vec = lax.iota(jnp.int32, _LANES)
                idx = jnp.where(r_vec < m, t0 + r_vec, T)
                pltpu.async_copy(tab_hbm.at[idx], bufs[0], semB).wait()
                pltpu.sync_copy(bufs[0], out_hbm.at[pl.ds(o0, C)])
            return carry

        lax.fori_loop(0, my_chunks, boundary, 0)

        def drain(j, carry):
            pltpu.make_async_copy(
                zbuf, out_hbm.at[pl.ds(wid * C, C)], semZ).wait()
            return carry

        lax.fori_loop(0, npad, drain, 0)

    return _k(input_len, table2z)


def _make_shift_body(n_shift):
    def _shift_body(a_ref, b_ref, o_ref):
        @pl.when(pl.program_id(0) < n_shift)
        def _():
            o_ref[...] = jnp.concatenate([a_ref[1:], b_ref[:1]], axis=0)

        @pl.when(pl.program_id(0) == n_shift)
        def _():
            o_ref[...] = jnp.zeros_like(o_ref)

    return _shift_body


@jax.jit
def _shift_table(pos_table):
    V, D = pos_table.shape
    T = V - 1
    CB = 2048
    n_shift = T // CB
    nb8 = (V - 1) // 8
    return pl.pallas_call(
        _make_shift_body(n_shift),
        grid=(n_shift + 1,),
        in_specs=[
            pl.BlockSpec((CB, D), lambda r: (jnp.minimum(r, n_shift - 1), 0)),
            pl.BlockSpec((8, D),
                         lambda r: (jnp.minimum((r + 1) * (CB // 8), nb8), 0)),
        ],
        out_specs=pl.BlockSpec((CB, D), lambda r: (r, 0)),
        out_shape=jax.ShapeDtypeStruct((T + 16, D), jnp.float32),
    )(pos_table, pos_table)


def kernel(input_len, max_len, pos_table):
    del max_len
    V, D = pos_table.shape
    T = V - 1
    B = input_len.shape[0]
    table2z = _shift_table(pos_table)
    out = _sc_expand(input_len, table2z, B, T, D)
    return out.reshape(B, T, D)
